# baseline (speedup 1.0000x reference)
import functools

import jax
import jax.numpy as jnp
from jax import lax
from jax.experimental import pallas as pl
from jax.experimental.pallas import tpu as pltpu
from jax.experimental.pallas import tpu_sc as plsc

BATCH = 4096
INPUT_SIZE = 256
HIDDEN_SIZE = 8192
OUTPUT_SIZE = 1000
D_PAD = 1024

BB = 1024
BH = 2048
NB = BATCH // BB
NH = HIDDEN_SIZE // BH


def _winner_body(x_ref, kw_ref, x2_ref, w2_ref, gw_ref,
                 w_ref, table_ref, bestv_ref, besti_ref):
    j = pl.program_id(0)
    i = pl.program_id(1)
    s = lax.dot_general(x_ref[...], kw_ref[...], (((1,), (1,)), ((), ())),
                        preferred_element_type=jnp.float32)
    val = (x2_ref[...] + w2_ref[...]) - 2.0 * s
    m = jnp.min(val, axis=1, keepdims=True)
    ids = lax.broadcasted_iota(jnp.int32, (BB, BH), 1)
    idx = jnp.min(jnp.where(val == m, ids, jnp.int32(2**30)),
                  axis=1, keepdims=True) + j * BH

    rows = pl.ds(i * BB, BB)

    @pl.when(j == 0)
    def _():
        bestv_ref[rows, :] = m
        besti_ref[rows, :] = idx

    @pl.when(j > 0)
    def _():
        prev = bestv_ref[rows, :]
        take = m < prev
        bestv_ref[rows, :] = jnp.where(take, m, prev)
        besti_ref[rows, :] = jnp.where(take, idx, besti_ref[rows, :])

    @pl.when(j == NH - 1)
    def _():
        w_ref[...] = besti_ref[rows, :]

    @pl.when(i == 0)
    def _():
        table_ref[:, :OUTPUT_SIZE] = jnp.transpose(gw_ref[...])


def _winners_and_table(x, kw, gw):
    x2 = jnp.sum(x * x, axis=1, keepdims=True)
    w2 = jnp.sum(kw * kw, axis=1)[None, :]
    winners, table = pl.pallas_call(
        _winner_body,
        grid=(NH, NB),
        in_specs=[
            pl.BlockSpec((BB, INPUT_SIZE), lambda j, i: (i, 0)),
            pl.BlockSpec((BH, INPUT_SIZE), lambda j, i: (j, 0)),
            pl.BlockSpec((BB, 1), lambda j, i: (i, 0)),
            pl.BlockSpec((1, BH), lambda j, i: (0, j)),
            pl.BlockSpec((OUTPUT_SIZE, BH), lambda j, i: (0, j)),
        ],
        out_specs=[
            pl.BlockSpec((BB, 1), lambda j, i: (i, 0)),
            pl.BlockSpec((BH, D_PAD), lambda j, i: (j, 0)),
        ],
        out_shape=[
            jax.ShapeDtypeStruct((BATCH, 1), jnp.int32),
            jax.ShapeDtypeStruct((HIDDEN_SIZE, D_PAD), jnp.float32),
        ],
        scratch_shapes=[
            pltpu.VMEM((BATCH, 1), jnp.float32),
            pltpu.VMEM((BATCH, 1), jnp.int32),
        ],
        compiler_params=pltpu.CompilerParams(
            dimension_semantics=("arbitrary", "parallel"),
        ),
    )(x, kw, x2, w2, gw)
    return winners.reshape(BATCH), table


@functools.lru_cache(maxsize=None)
def _make_gather():
    info = plsc.get_sparse_core_info()
    nc, ns = info.num_cores, info.num_subcores
    nw = nc * ns
    b_per_w = BATCH // nw

    ch = 32
    nch = b_per_w // ch

    mesh = plsc.VectorSubcoreMesh(core_axis_name="c", subcore_axis_name="s")

    @functools.partial(
        pl.kernel, mesh=mesh,
        out_type=jax.ShapeDtypeStruct((BATCH, D_PAD), jnp.float32),
        scratch_types=[
            pltpu.VMEM((nch, ch), jnp.int32),
            pltpu.VMEM((ch, D_PAD), jnp.float32),
            pltpu.VMEM((ch, D_PAD), jnp.float32),
            pltpu.SemaphoreType.DMA,
            pltpu.SemaphoreType.DMA,
        ],
    )
    def gather_k(table_hbm, idx_hbm, out_hbm, idx_v, buf_a, buf_b, sem_a, sem_b):
        wid = lax.axis_index("s") * nc + lax.axis_index("c")
        base = wid * b_per_w
        for t in range(nch):
            pltpu.sync_copy(idx_hbm.at[pl.ds(base + t * ch, ch)], idx_v.at[t])
        bufs = (buf_a, buf_b)
        sems = (sem_a, sem_b)
        pending = [None, None]
        pending[0] = pltpu.async_copy(table_hbm.at[idx_v.at[0]], bufs[0], sems[0])
        for t in range(nch):
            if t + 1 < nch:
                pending[(t + 1) % 2] = pltpu.async_copy(
                    table_hbm.at[idx_v.at[t + 1]], bufs[(t + 1) % 2],
                    sems[(t + 1) % 2])
            pending[t % 2].wait()
            pltpu.sync_copy(bufs[t % 2], out_hbm.at[pl.ds(base + t * ch, ch)])

    return gather_k


def kernel(x, kohonen_weights, grossberg_weights):
    winners, table = _winners_and_table(
        x.reshape(x.shape[0], -1), kohonen_weights, grossberg_weights)
    out = _make_gather()(table, winners)
    return out[:, :OUTPUT_SIZE], winners


# speedup vs baseline: 1.0048x; 1.0048x over previous
import functools

import jax
import jax.numpy as jnp
from jax import lax
from jax.experimental import pallas as pl
from jax.experimental.pallas import tpu as pltpu
from jax.experimental.pallas import tpu_sc as plsc

BATCH = 4096
INPUT_SIZE = 256
HIDDEN_SIZE = 8192
OUTPUT_SIZE = 1000
D_PAD = 1024

BB = 1024
BH = 2048
NB = BATCH // BB
NH = HIDDEN_SIZE // BH


def _winner_body(x_ref, kw_ref, x2_ref, w2_ref, gw_ref,
                 w_ref, table_ref, bestv_ref, besti_ref):
    j = pl.program_id(0)
    i = pl.program_id(1)
    s = lax.dot_general(x_ref[...], kw_ref[...], (((1,), (1,)), ((), ())),
                        preferred_element_type=jnp.float32)
    val = (x2_ref[...] + w2_ref[...]) - 2.0 * s
    m = jnp.min(val, axis=1, keepdims=True)
    ids = lax.broadcasted_iota(jnp.int32, (BB, BH), 1)
    idx = jnp.min(jnp.where(val == m, ids, jnp.int32(2**30)),
                  axis=1, keepdims=True) + j * BH

    rows = pl.ds(i * BB, BB)

    @pl.when(j == 0)
    def _():
        bestv_ref[rows, :] = m
        besti_ref[rows, :] = idx

    @pl.when(j > 0)
    def _():
        prev = bestv_ref[rows, :]
        take = m < prev
        bestv_ref[rows, :] = jnp.where(take, m, prev)
        besti_ref[rows, :] = jnp.where(take, idx, besti_ref[rows, :])

    @pl.when(j == NH - 1)
    def _():
        w_ref[...] = besti_ref[rows, :]

    @pl.when(i == 0)
    def _():
        table_ref[:, :OUTPUT_SIZE] = jnp.transpose(gw_ref[...])


def _winners_and_table(x, kw, gw):
    x2 = jnp.sum(x * x, axis=1, keepdims=True)
    w2 = jnp.sum(kw * kw, axis=1)[None, :]
    winners, table = pl.pallas_call(
        _winner_body,
        grid=(NH, NB),
        in_specs=[
            pl.BlockSpec((BB, INPUT_SIZE), lambda j, i: (i, 0)),
            pl.BlockSpec((BH, INPUT_SIZE), lambda j, i: (j, 0)),
            pl.BlockSpec((BB, 1), lambda j, i: (i, 0)),
            pl.BlockSpec((1, BH), lambda j, i: (0, j)),
            pl.BlockSpec((OUTPUT_SIZE, BH), lambda j, i: (0, j)),
        ],
        out_specs=[
            pl.BlockSpec((BB, 1), lambda j, i: (i, 0)),
            pl.BlockSpec((BH, D_PAD), lambda j, i: (j, 0)),
        ],
        out_shape=[
            jax.ShapeDtypeStruct((BATCH, 1), jnp.int32),
            jax.ShapeDtypeStruct((HIDDEN_SIZE, D_PAD), jnp.float32),
        ],
        scratch_shapes=[
            pltpu.VMEM((BATCH, 1), jnp.float32),
            pltpu.VMEM((BATCH, 1), jnp.int32),
        ],
    )(x, kw, x2, w2, gw)
    return winners.reshape(BATCH), table


@functools.lru_cache(maxsize=None)
def _make_gather():
    info = plsc.get_sparse_core_info()
    nc, ns = info.num_cores, info.num_subcores
    nw = nc * ns
    b_per_w = BATCH // nw

    ch = 32
    nch = b_per_w // ch

    mesh = plsc.VectorSubcoreMesh(core_axis_name="c", subcore_axis_name="s")

    @functools.partial(
        pl.kernel, mesh=mesh,
        out_type=jax.ShapeDtypeStruct((BATCH, D_PAD), jnp.float32),
        scratch_types=[
            pltpu.VMEM((nch, ch), jnp.int32),
            pltpu.VMEM((ch, D_PAD), jnp.float32),
            pltpu.VMEM((ch, D_PAD), jnp.float32),
            pltpu.SemaphoreType.DMA,
            pltpu.SemaphoreType.DMA,
        ],
    )
    def gather_k(table_hbm, idx_hbm, out_hbm, idx_v, buf_a, buf_b, sem_a, sem_b):
        wid = lax.axis_index("s") * nc + lax.axis_index("c")
        base = wid * b_per_w
        for t in range(nch):
            pltpu.sync_copy(idx_hbm.at[pl.ds(base + t * ch, ch)], idx_v.at[t])
        bufs = (buf_a, buf_b)
        sems = (sem_a, sem_b)
        pending = [None, None]
        pending[0] = pltpu.async_copy(table_hbm.at[idx_v.at[0]], bufs[0], sems[0])
        for t in range(nch):
            if t + 1 < nch:
                pending[(t + 1) % 2] = pltpu.async_copy(
                    table_hbm.at[idx_v.at[t + 1]], bufs[(t + 1) % 2],
                    sems[(t + 1) % 2])
            pending[t % 2].wait()
            pltpu.sync_copy(bufs[t % 2], out_hbm.at[pl.ds(base + t * ch, ch)])

    return gather_k


def kernel(x, kohonen_weights, grossberg_weights):
    winners, table = _winners_and_table(
        x.reshape(x.shape[0], -1), kohonen_weights, grossberg_weights)
    out = _make_gather()(table, winners)
    return out[:, :OUTPUT_SIZE], winners
